# Initial kernel scaffold; baseline (speedup 1.0000x reference)
import jax
import jax.numpy as jnp
from jax.experimental import pallas as pl


def kernel(x, edge_index, W1, b1, W2, b2, Wf1, bf1, Wf2, bf2):
    raise NotImplementedError("write your pallas kernel here")


# speedup vs baseline: 6.8067x; 6.8067x over previous
import functools

import jax
import jax.numpy as jnp
from jax import lax
from jax.experimental import pallas as pl
from jax.experimental.pallas import tpu as pltpu
from jax.experimental.pallas import tpu_sc as plsc

N = 10000
NP = 10240
E = 320000
EP = 327680
NC = 2
NS = 16
NW = NC * NS
EPT = EP // NW
B = 128
NB = EPT // B
RPT = NP // NS
D_IN = 128
D_HID = 256
GB = 1024
NG = NP // GB

_mesh = plsc.VectorSubcoreMesh(
    core_axis_name="c", subcore_axis_name="s", num_cores=NC, num_subcores=NS
)


def _deg_body(dst_hbm, out_hbm, dst_v, buf_v, acc_sh):
    c = lax.axis_index("c")
    s = lax.axis_index("s")
    wid = s * NC + c

    fz = jnp.zeros((16,), jnp.float32)
    fo = jnp.ones((16,), jnp.float32)

    def zloop(i, _):
        buf_v[i // 8, pl.ds((i % 8) * 16, 16)] = fz
        return 0

    lax.fori_loop(0, B * D_IN // 16, zloop, 0)

    for j in range(RPT // B):
        pltpu.sync_copy(buf_v, acc_sh.at[pl.ds(s * RPT + j * B, B)])

    def oloop(i, _):
        buf_v[i // 8, pl.ds((i % 8) * 16, 16)] = fo
        return 0

    lax.fori_loop(0, B * D_IN // 16, oloop, 0)

    pltpu.sync_copy(dst_hbm.at[wid], dst_v)
    plsc.subcore_barrier()

    def dloop(b, _):
        pltpu.sync_copy(buf_v, acc_sh.at[dst_v.at[b]], add=True)
        return 0

    lax.fori_loop(0, NB, dloop, 0)
    plsc.subcore_barrier()

    pltpu.sync_copy(acc_sh.at[pl.ds(s * RPT, RPT)], out_hbm.at[c, pl.ds(s * RPT, RPT)])


_deg_call = pl.kernel(
    _deg_body,
    out_type=jax.ShapeDtypeStruct((NC, NP, D_IN), jnp.float32),
    mesh=_mesh,
    scratch_types=[
        pltpu.VMEM((NB, B), jnp.int32),
        pltpu.VMEM((B, D_IN), jnp.float32),
        pltpu.VMEM_SHARED((NP, D_IN), jnp.float32),
    ],
)


def _seg_body(table_hbm, src_hbm, dst_hbm, out_hbm, src_v, dst_v, buf_v, acc_sh):
    c = lax.axis_index("c")
    s = lax.axis_index("s")
    wid = s * NC + c

    fz = jnp.zeros((16,), jnp.float32)

    def zloop(i, _):
        buf_v[i // 8, pl.ds((i % 8) * 16, 16)] = fz
        return 0

    lax.fori_loop(0, B * D_IN // 16, zloop, 0)

    for j in range(RPT // B):
        pltpu.sync_copy(buf_v, acc_sh.at[pl.ds(s * RPT + j * B, B)])

    pltpu.sync_copy(src_hbm.at[wid], src_v)
    pltpu.sync_copy(dst_hbm.at[wid], dst_v)
    plsc.subcore_barrier()

    def body(b, _):
        pltpu.sync_copy(table_hbm.at[src_v.at[b]], buf_v)
        pltpu.sync_copy(buf_v, acc_sh.at[dst_v.at[b]], add=True)
        return 0

    lax.fori_loop(0, NB, body, 0)
    plsc.subcore_barrier()

    pltpu.sync_copy(acc_sh.at[pl.ds(s * RPT, RPT)], out_hbm.at[c, pl.ds(s * RPT, RPT)])


_seg_call = pl.kernel(
    _seg_body,
    out_type=jax.ShapeDtypeStruct((NC, NP, D_IN), jnp.float32),
    mesh=_mesh,
    scratch_types=[
        pltpu.VMEM((NB, B), jnp.int32),
        pltpu.VMEM((NB, B), jnp.int32),
        pltpu.VMEM((B, D_IN), jnp.float32),
        pltpu.VMEM_SHARED((NP, D_IN), jnp.float32),
    ],
)


def _prep_body(dp_ref, x_ref, dinv_ref, xs_ref):
    deg = dp_ref[0][:, :1] + dp_ref[1][:, :1] + 1.0
    dinv = lax.rsqrt(deg)
    dinv_ref[...] = dinv
    xs_ref[...] = x_ref[...] * dinv


_prep_call = pl.pallas_call(
    _prep_body,
    grid=(NG,),
    in_specs=[
        pl.BlockSpec((NC, GB, D_IN), lambda i: (0, i, 0)),
        pl.BlockSpec((GB, D_IN), lambda i: (i, 0)),
    ],
    out_specs=[
        pl.BlockSpec((GB, 1), lambda i: (i, 0)),
        pl.BlockSpec((GB, D_IN), lambda i: (i, 0)),
    ],
    out_shape=[
        jax.ShapeDtypeStruct((NP, 1), jnp.float32),
        jax.ShapeDtypeStruct((NP, D_IN), jnp.float32),
    ],
)


def _mid_body(aggp_ref, xs_ref, dinv_ref, w1_ref, b1_ref, t0_ref, t1_ref):
    agg = aggp_ref[0] + aggp_ref[1] + xs_ref[...]
    pre = agg * dinv_ref[...]
    h = jnp.dot(pre, w1_ref[...], preferred_element_type=jnp.float32) + b1_ref[...]
    t = jnp.maximum(h, 0.0) * dinv_ref[...]
    t0_ref[...] = t[:, :D_IN]
    t1_ref[...] = t[:, D_IN:]


_mid_call = pl.pallas_call(
    _mid_body,
    grid=(NG,),
    in_specs=[
        pl.BlockSpec((NC, GB, D_IN), lambda i: (0, i, 0)),
        pl.BlockSpec((GB, D_IN), lambda i: (i, 0)),
        pl.BlockSpec((GB, 1), lambda i: (i, 0)),
        pl.BlockSpec((D_IN, D_HID), lambda i: (0, 0)),
        pl.BlockSpec((1, D_HID), lambda i: (0, 0)),
    ],
    out_specs=[
        pl.BlockSpec((GB, D_IN), lambda i: (i, 0)),
        pl.BlockSpec((GB, D_IN), lambda i: (i, 0)),
    ],
    out_shape=[
        jax.ShapeDtypeStruct((NP, D_IN), jnp.float32),
        jax.ShapeDtypeStruct((NP, D_IN), jnp.float32),
    ],
)


def _head_body(a0_ref, a1_ref, t0_ref, t1_ref, dinv_ref, w2_ref, b2_ref,
               wf1_ref, bf1_ref, wf2_ref, bf2_ref, o_ref):
    a0 = a0_ref[0] + a0_ref[1] + t0_ref[...]
    a1 = a1_ref[0] + a1_ref[1] + t1_ref[...]
    agg = jnp.concatenate([a0, a1], axis=1) * dinv_ref[...]
    h2 = jnp.dot(agg, w2_ref[...], preferred_element_type=jnp.float32) + b2_ref[...]
    h2 = jnp.maximum(h2, 0.0)
    h3 = jnp.dot(h2, wf1_ref[...], preferred_element_type=jnp.float32) + bf1_ref[...]
    h3 = jnp.maximum(h3, 0.0)
    o_ref[...] = jnp.dot(h3, wf2_ref[...], preferred_element_type=jnp.float32) + bf2_ref[...]


_head_call = pl.pallas_call(
    _head_body,
    grid=(NG,),
    in_specs=[
        pl.BlockSpec((NC, GB, D_IN), lambda i: (0, i, 0)),
        pl.BlockSpec((NC, GB, D_IN), lambda i: (0, i, 0)),
        pl.BlockSpec((GB, D_IN), lambda i: (i, 0)),
        pl.BlockSpec((GB, D_IN), lambda i: (i, 0)),
        pl.BlockSpec((GB, 1), lambda i: (i, 0)),
        pl.BlockSpec((D_HID, D_HID), lambda i: (0, 0)),
        pl.BlockSpec((1, D_HID), lambda i: (0, 0)),
        pl.BlockSpec((D_HID, D_HID // 2), lambda i: (0, 0)),
        pl.BlockSpec((1, D_HID // 2), lambda i: (0, 0)),
        pl.BlockSpec((D_HID // 2, 1), lambda i: (0, 0)),
        pl.BlockSpec((1, 1), lambda i: (0, 0)),
    ],
    out_specs=pl.BlockSpec((GB, 1), lambda i: (i, 0)),
    out_shape=jax.ShapeDtypeStruct((NP, 1), jnp.float32),
)


def kernel(x, edge_index, W1, b1, W2, b2, Wf1, bf1, Wf2, bf2):
    ei = edge_index.astype(jnp.int32)
    pad = jnp.full((EP - E,), N, jnp.int32)
    srcp = jnp.concatenate([ei[0], pad]).reshape(NW, NB, B)
    dstp = jnp.concatenate([ei[1], pad]).reshape(NW, NB, B)
    x_pad = jnp.zeros((NP, D_IN), jnp.float32).at[:N].set(x)

    degp = _deg_call(dstp)
    dinv, xs = _prep_call(degp, x_pad)
    aggp1 = _seg_call(xs, srcp, dstp)
    t0, t1 = _mid_call(aggp1, xs, dinv, W1, b1.reshape(1, -1))
    a0p = _seg_call(t0, srcp, dstp)
    a1p = _seg_call(t1, srcp, dstp)
    o = _head_call(a0p, a1p, t0, t1, dinv, W2, b2.reshape(1, -1),
                   Wf1, bf1.reshape(1, -1), Wf2, bf2.reshape(1, -1))
    return o[:N, 0]


# speedup vs baseline: 17.4701x; 2.5666x over previous
import functools

import jax
import jax.numpy as jnp
from jax import lax
from jax.experimental import pallas as pl
from jax.experimental.pallas import tpu as pltpu
from jax.experimental.pallas import tpu_sc as plsc

N = 10000
NP = 10240
E = 320000
EP = 327680
NC = 2
NS = 16
NW = NC * NS
EPT = EP // NW
B = 128
NB = EPT // B
RPT = NP // NS
D_IN = 128
D_HID = 256
GB = 1024
NG = NP // GB

_mesh = plsc.VectorSubcoreMesh(
    core_axis_name="c", subcore_axis_name="s", num_cores=NC, num_subcores=NS
)


def _deg_body(dst_hbm, out_hbm, dst_v, buf_v, acc_sh):
    c = lax.axis_index("c")
    s = lax.axis_index("s")
    wid = s * NC + c

    fz = jnp.zeros((16,), jnp.float32)
    fo = jnp.ones((16,), jnp.float32)

    def zloop(i, _):
        buf_v[i // 8, pl.ds((i % 8) * 16, 16)] = fz
        return 0

    lax.fori_loop(0, B * D_IN // 16, zloop, 0)

    for j in range(RPT // B):
        pltpu.sync_copy(buf_v, acc_sh.at[pl.ds(s * RPT + j * B, B)])

    def oloop(i, _):
        buf_v[i // 8, pl.ds((i % 8) * 16, 16)] = fo
        return 0

    lax.fori_loop(0, B * D_IN // 16, oloop, 0)

    pltpu.sync_copy(dst_hbm.at[wid], dst_v)
    plsc.subcore_barrier()

    def dloop(b, _):
        pltpu.sync_copy(buf_v, acc_sh.at[dst_v.at[b]], add=True)
        return 0

    lax.fori_loop(0, NB, dloop, 0)
    plsc.subcore_barrier()

    pltpu.sync_copy(acc_sh.at[pl.ds(s * RPT, RPT)], out_hbm.at[c, pl.ds(s * RPT, RPT)])


_deg_call = pl.kernel(
    _deg_body,
    out_type=jax.ShapeDtypeStruct((NC, NP, D_IN), jnp.float32),
    mesh=_mesh,
    scratch_types=[
        pltpu.VMEM((NB, B), jnp.int32),
        pltpu.VMEM((B, D_IN), jnp.float32),
        pltpu.VMEM_SHARED((NP, D_IN), jnp.float32),
    ],
)


def _seg_body(table_hbm, src_hbm, dst_hbm, out_hbm, src_v, dst_v, buf_v, acc_sh):
    c = lax.axis_index("c")
    s = lax.axis_index("s")
    wid = s * NC + c

    fz = jnp.zeros((16,), jnp.float32)

    def zloop(i, _):
        buf_v[i // 8, pl.ds((i % 8) * 16, 16)] = fz
        return 0

    lax.fori_loop(0, B * D_IN // 16, zloop, 0)

    for j in range(RPT // B):
        pltpu.sync_copy(buf_v, acc_sh.at[pl.ds(s * RPT + j * B, B)])

    pltpu.sync_copy(src_hbm.at[wid], src_v)
    pltpu.sync_copy(dst_hbm.at[wid], dst_v)
    plsc.subcore_barrier()

    def body(b, _):
        pltpu.sync_copy(table_hbm.at[src_v.at[b]], buf_v)
        pltpu.sync_copy(buf_v, acc_sh.at[dst_v.at[b]], add=True)
        return 0

    lax.fori_loop(0, NB, body, 0)
    plsc.subcore_barrier()

    pltpu.sync_copy(acc_sh.at[pl.ds(s * RPT, RPT)], out_hbm.at[c, pl.ds(s * RPT, RPT)])


_seg_call = pl.kernel(
    _seg_body,
    out_type=jax.ShapeDtypeStruct((NC, NP, D_IN), jnp.float32),
    mesh=_mesh,
    scratch_types=[
        pltpu.VMEM((NB, B), jnp.int32),
        pltpu.VMEM((NB, B), jnp.int32),
        pltpu.VMEM((B, D_IN), jnp.float32),
        pltpu.VMEM_SHARED((NP, D_IN), jnp.float32),
    ],
)


def _prep_body(dp_ref, x_ref, dinv_ref, xs_ref):
    deg = dp_ref[0][:, :1] + dp_ref[1][:, :1] + 1.0
    dinv = lax.rsqrt(deg)
    dinv_ref[...] = dinv
    xs_ref[...] = x_ref[...] * dinv


_prep_call = pl.pallas_call(
    _prep_body,
    grid=(NG,),
    in_specs=[
        pl.BlockSpec((NC, GB, D_IN), lambda i: (0, i, 0)),
        pl.BlockSpec((GB, D_IN), lambda i: (i, 0)),
    ],
    out_specs=[
        pl.BlockSpec((GB, 1), lambda i: (i, 0)),
        pl.BlockSpec((GB, D_IN), lambda i: (i, 0)),
    ],
    out_shape=[
        jax.ShapeDtypeStruct((NP, 1), jnp.float32),
        jax.ShapeDtypeStruct((NP, D_IN), jnp.float32),
    ],
)


def _mid_body(aggp_ref, xs_ref, dinv_ref, w1_ref, b1_ref, t0_ref, t1_ref):
    agg = aggp_ref[0] + aggp_ref[1] + xs_ref[...]
    pre = agg * dinv_ref[...]
    h = jnp.dot(pre, w1_ref[...], preferred_element_type=jnp.float32) + b1_ref[...]
    t = jnp.maximum(h, 0.0) * dinv_ref[...]
    t0_ref[...] = t[:, :D_IN]
    t1_ref[...] = t[:, D_IN:]


_mid_call = pl.pallas_call(
    _mid_body,
    grid=(NG,),
    in_specs=[
        pl.BlockSpec((NC, GB, D_IN), lambda i: (0, i, 0)),
        pl.BlockSpec((GB, D_IN), lambda i: (i, 0)),
        pl.BlockSpec((GB, 1), lambda i: (i, 0)),
        pl.BlockSpec((D_IN, D_HID), lambda i: (0, 0)),
        pl.BlockSpec((1, D_HID), lambda i: (0, 0)),
    ],
    out_specs=[
        pl.BlockSpec((GB, D_IN), lambda i: (i, 0)),
        pl.BlockSpec((GB, D_IN), lambda i: (i, 0)),
    ],
    out_shape=[
        jax.ShapeDtypeStruct((NP, D_IN), jnp.float32),
        jax.ShapeDtypeStruct((NP, D_IN), jnp.float32),
    ],
)


def _head_body(a0_ref, a1_ref, t0_ref, t1_ref, dinv_ref, w2_ref, b2_ref,
               wf1_ref, bf1_ref, wf2_ref, bf2_ref, o_ref):
    a0 = a0_ref[0] + a0_ref[1] + t0_ref[...]
    a1 = a1_ref[0] + a1_ref[1] + t1_ref[...]
    agg = jnp.concatenate([a0, a1], axis=1) * dinv_ref[...]
    h2 = jnp.dot(agg, w2_ref[...], preferred_element_type=jnp.float32) + b2_ref[...]
    h2 = jnp.maximum(h2, 0.0)
    h3 = jnp.dot(h2, wf1_ref[...], preferred_element_type=jnp.float32) + bf1_ref[...]
    h3 = jnp.maximum(h3, 0.0)
    o_ref[...] = jnp.dot(h3, wf2_ref[...], preferred_element_type=jnp.float32) + bf2_ref[...]


_head_call = pl.pallas_call(
    _head_body,
    grid=(NG,),
    in_specs=[
        pl.BlockSpec((NC, GB, D_IN), lambda i: (0, i, 0)),
        pl.BlockSpec((NC, GB, D_IN), lambda i: (0, i, 0)),
        pl.BlockSpec((GB, D_IN), lambda i: (i, 0)),
        pl.BlockSpec((GB, D_IN), lambda i: (i, 0)),
        pl.BlockSpec((GB, 1), lambda i: (i, 0)),
        pl.BlockSpec((D_HID, D_HID), lambda i: (0, 0)),
        pl.BlockSpec((1, D_HID), lambda i: (0, 0)),
        pl.BlockSpec((D_HID, D_HID // 2), lambda i: (0, 0)),
        pl.BlockSpec((1, D_HID // 2), lambda i: (0, 0)),
        pl.BlockSpec((D_HID // 2, 1), lambda i: (0, 0)),
        pl.BlockSpec((1, 1), lambda i: (0, 0)),
    ],
    out_specs=pl.BlockSpec((GB, 1), lambda i: (i, 0)),
    out_shape=jax.ShapeDtypeStruct((NP, 1), jnp.float32),
)


def kernel(x, edge_index, W1, b1, W2, b2, Wf1, bf1, Wf2, bf2):
    ei = edge_index.astype(jnp.int32)
    pad = N + (jnp.arange(EP - E, dtype=jnp.int32) % (NP - N))
    srcp = jnp.concatenate([ei[0], pad]).reshape(NW, NB, B)
    dstp = jnp.concatenate([ei[1], pad]).reshape(NW, NB, B)
    x_pad = jnp.zeros((NP, D_IN), jnp.float32).at[:N].set(x)

    degp = _deg_call(dstp)
    dinv, xs = _prep_call(degp, x_pad)
    aggp1 = _seg_call(xs, srcp, dstp)
    t0, t1 = _mid_call(aggp1, xs, dinv, W1, b1.reshape(1, -1))
    a0p = _seg_call(t0, srcp, dstp)
    a1p = _seg_call(t1, srcp, dstp)
    o = _head_call(a0p, a1p, t0, t1, dinv, W2, b2.reshape(1, -1),
                   Wf1, bf1.reshape(1, -1), Wf2, bf2.reshape(1, -1))
    return o[:N, 0]
